# Initial kernel scaffold; baseline (speedup 1.0000x reference)
import jax
import jax.numpy as jnp
from jax.experimental import pallas as pl


def kernel(atom_features, edges_features, state_attrs, pair_indices, atom_graph_indices, bond_graph_indices, W_e, U_e, b_e, W_n, U_n, b_n, W_s, U_s, b_s, nodes_kernel, nodes_bias):
    raise NotImplementedError("write your pallas kernel here")


# speedup vs baseline: 1.6780x; 1.6780x over previous
import functools

import jax
import jax.numpy as jnp
from jax import lax
from jax.experimental import pallas as pl
from jax.experimental.pallas import tpu as pltpu
from jax.experimental.pallas import tpu_sc as plsc

_N = 10000
_E = 160000
_G = 128
_AD = 16
_ED = 16
_SD = 8
_D = _AD + _SD
_STEPS = 2

_NPAD = 10240
_BN = 1024
_BE = 1000

_NW = 32
_EPW = _E // _NW
_KCH = 1000
_NSH = _NPAD // 16

_F32 = jnp.float32


def _gru_math(mx, mh, h):
    u = mx.shape[-1] // 3
    z = jax.nn.sigmoid(mx[:, :u] + mh[:, :u])
    r = jax.nn.sigmoid(mx[:, u:2 * u] + mh[:, u:2 * u])
    hh = jnp.tanh(mx[:, 2 * u:] + r * mh[:, 2 * u:])
    return z * h + (1.0 - z) * hh


def _sc_gather(src, dst, af, amsf):
    mesh = plsc.VectorSubcoreMesh(core_axis_name="c", subcore_axis_name="s")

    @functools.partial(
        pl.kernel,
        out_type=[
            jax.ShapeDtypeStruct((_E, _AD), _F32),
            jax.ShapeDtypeStruct((_E, _D), _F32),
        ],
        mesh=mesh,
        scratch_types=[
            pltpu.VMEM((_KCH,), jnp.int32),
            pltpu.VMEM((_KCH,), jnp.int32),
            pltpu.VMEM((_KCH, _AD), _F32),
            pltpu.VMEM((_KCH, _D), _F32),
            pltpu.SemaphoreType.DMA,
            pltpu.SemaphoreType.DMA,
        ],
        compiler_params=pltpu.CompilerParams(use_tc_tiling_on_sc=False),
    )
    def run(src_h, dst_h, af_h, amsf_h, a0_h, nm_h,
            isrc, idst, abuf, nbuf, sem_a, sem_n):
        wid = lax.axis_index("s") * 2 + lax.axis_index("c")
        base0 = wid * _EPW
        for t in range(_EPW // _KCH):
            base = base0 + t * _KCH
            pltpu.sync_copy(src_h.at[pl.ds(base, _KCH)], isrc)
            pltpu.sync_copy(dst_h.at[pl.ds(base, _KCH)], idst)
            cp_a = pltpu.async_copy(af_h.at[isrc], abuf, sem_a)
            cp_n = pltpu.async_copy(amsf_h.at[idst], nbuf, sem_n)
            cp_a.wait()
            cp_n.wait()
            pltpu.sync_copy(abuf, a0_h.at[pl.ds(base, _KCH)])
            pltpu.sync_copy(nbuf, nm_h.at[pl.ds(base, _KCH)])

    return run(src, dst, af, amsf)


def _sc_scatter(src, trans, zrows):
    mesh = plsc.VectorSubcoreMesh(core_axis_name="c", subcore_axis_name="s")

    @functools.partial(
        pl.kernel,
        out_type=jax.ShapeDtypeStruct((2, _NPAD, _D), _F32),
        mesh=mesh,
        scratch_types=[
            pltpu.VMEM((_KCH,), jnp.int32),
            pltpu.VMEM((_KCH, _D), _F32),
            pltpu.VMEM_SHARED((_NPAD, _D), _F32),
        ],
        compiler_params=pltpu.CompilerParams(use_tc_tiling_on_sc=False),
    )
    def run(src_h, tr_h, z_h, out_h, ibuf, tbuf, acc):
        c = lax.axis_index("c")
        s = lax.axis_index("s")
        pltpu.sync_copy(z_h, acc.at[pl.ds(s * _NSH, _NSH)])
        plsc.subcore_barrier()
        base0 = (c * 16 + s) * _EPW
        for t in range(_EPW // _KCH):
            base = base0 + t * _KCH
            pltpu.sync_copy(src_h.at[pl.ds(base, _KCH)], ibuf)
            pltpu.sync_copy(tr_h.at[pl.ds(base, _KCH)], tbuf)
            pltpu.sync_copy(tbuf, acc.at[ibuf], add=True)
        plsc.subcore_barrier()
        pltpu.sync_copy(acc.at[pl.ds(s * _NSH, _NSH)],
                        out_h.at[c, pl.ds(s * _NSH, _NSH)])

    return run(src, trans, zrows)


def _prep_body(ag_ref, af_ref, st_ref, out_ref):
    oh = (ag_ref[...] == lax.broadcasted_iota(jnp.int32, (1, _G), 1))
    strep = jnp.dot(oh.astype(_F32), st_ref[...], preferred_element_type=_F32, precision=lax.Precision.HIGHEST)
    out_ref[:, :_AD] = af_ref[...]
    out_ref[:, _AD:] = strep


def _prep_call(ag2, af, st):
    return pl.pallas_call(
        _prep_body,
        grid=(_NPAD // _BN,),
        in_specs=[
            pl.BlockSpec((_BN, 1), lambda i: (i, 0)),
            pl.BlockSpec((_BN, _AD), lambda i: (i, 0)),
            pl.BlockSpec((_G, _SD), lambda i: (0, 0)),
        ],
        out_specs=pl.BlockSpec((_BN, _D), lambda i: (i, 0)),
        out_shape=jax.ShapeDtypeStruct((_NPAD, _D), _F32),
    )(ag2, af, st)


def _edge_body(bg_ref, a0_ref, nm_ref, ef_ref, st_ref, We_ref, Ue_ref,
               be_ref, nk_ref, nb_ref, tile_ref, sel_ref,
               efo_ref, tro_ref, esum_ref):
    oh = (bg_ref[...] == lax.broadcasted_iota(jnp.int32, (1, _G), 1))
    oh = oh.astype(_F32)
    st_e = jnp.dot(oh, st_ref[...], preferred_element_type=_F32, precision=lax.Precision.HIGHEST)
    a0 = a0_ref[...]
    nm = nm_ref[...]
    ef = ef_ref[...]
    a1 = nm[:, :_AD]
    W = We_ref[...]
    mx = (jnp.dot(a0, W[0:16], preferred_element_type=_F32)
          + jnp.dot(a1, W[16:32], preferred_element_type=_F32)
          + jnp.dot(st_e, W[32:40], preferred_element_type=_F32)
          + jnp.dot(ef, W[40:56], preferred_element_type=_F32)
          + be_ref[0:1, :])
    mh = jnp.dot(ef, Ue_ref[...], preferred_element_type=_F32) + be_ref[1:2, :]
    ef_new = _gru_math(mx, mh, ef)
    efo_ref[...] = ef_new
    ew = jnp.dot(ef_new, nk_ref[...], preferred_element_type=_F32) + nb_ref[...]
    nt = jnp.dot(nm, tile_ref[...], preferred_element_type=_F32,
                 precision=lax.Precision.HIGHEST)
    tr = jnp.dot(ew * nt, sel_ref[...], preferred_element_type=_F32,
                 precision=lax.Precision.HIGHEST)
    tro_ref[...] = tr
    part = lax.dot_general(oh, ef_new, (((0,), (0,)), ((), ())),
                           preferred_element_type=_F32, precision=lax.Precision.HIGHEST)

    @pl.when(pl.program_id(0) == 0)
    def _():
        esum_ref[...] = jnp.zeros_like(esum_ref)

    esum_ref[...] += part


def _edge_call(bg2, a0, nm, ef, st, W_e, U_e, b_e, nk, nb1, tile, sel):
    full = lambda r, c: pl.BlockSpec((r, c), lambda i: (0, 0))
    return pl.pallas_call(
        _edge_body,
        grid=(_E // _BE,),
        in_specs=[
            pl.BlockSpec((_BE, 1), lambda i: (i, 0)),
            pl.BlockSpec((_BE, _AD), lambda i: (i, 0)),
            pl.BlockSpec((_BE, _D), lambda i: (i, 0)),
            pl.BlockSpec((_BE, _ED), lambda i: (i, 0)),
            full(_G, _SD),
            full(2 * _AD + _SD + _ED, 3 * _ED),
            full(_ED, 3 * _ED),
            full(2, 3 * _ED),
            full(_ED, _D * _D),
            full(1, _D * _D),
            full(_D, _D * _D),
            full(_D * _D, _D),
        ],
        out_specs=[
            pl.BlockSpec((_BE, _ED), lambda i: (i, 0)),
            pl.BlockSpec((_BE, _D), lambda i: (i, 0)),
            pl.BlockSpec((_G, _ED), lambda i: (0, 0)),
        ],
        out_shape=[
            jax.ShapeDtypeStruct((_E, _ED), _F32),
            jax.ShapeDtypeStruct((_E, _D), _F32),
            jax.ShapeDtypeStruct((_G, _ED), _F32),
        ],
    )(bg2, a0, nm, ef, st, W_e, U_e, b_e, nk, nb1, tile, sel)


def _node_body(ag_ref, p0_ref, p1_ref, af_ref, Wn_ref, Un_ref, bn_ref,
               afo_ref, asum_ref):
    agg = p0_ref[...] + p1_ref[...]
    af = af_ref[...]
    mx = jnp.dot(agg, Wn_ref[...], preferred_element_type=_F32) + bn_ref[0:1, :]
    mh = jnp.dot(af, Un_ref[...], preferred_element_type=_F32) + bn_ref[1:2, :]
    af_new = _gru_math(mx, mh, af)
    afo_ref[...] = af_new
    oh = (ag_ref[...] == lax.broadcasted_iota(jnp.int32, (1, _G), 1))
    part = lax.dot_general(oh.astype(_F32), af_new, (((0,), (0,)), ((), ())),
                           preferred_element_type=_F32, precision=lax.Precision.HIGHEST)

    @pl.when(pl.program_id(0) == 0)
    def _():
        asum_ref[...] = jnp.zeros_like(asum_ref)

    asum_ref[...] += part


def _node_call(ag2, p0, p1, af, W_n, U_n, b_n):
    full = lambda r, c: pl.BlockSpec((r, c), lambda i: (0, 0))
    return pl.pallas_call(
        _node_body,
        grid=(_NPAD // _BN,),
        in_specs=[
            pl.BlockSpec((_BN, 1), lambda i: (i, 0)),
            pl.BlockSpec((_BN, _D), lambda i: (i, 0)),
            pl.BlockSpec((_BN, _D), lambda i: (i, 0)),
            pl.BlockSpec((_BN, _AD), lambda i: (i, 0)),
            full(_D, 3 * _AD),
            full(_AD, 3 * _AD),
            full(2, 3 * _AD),
        ],
        out_specs=[
            pl.BlockSpec((_BN, _AD), lambda i: (i, 0)),
            pl.BlockSpec((_G, _AD), lambda i: (0, 0)),
        ],
        out_shape=[
            jax.ShapeDtypeStruct((_NPAD, _AD), _F32),
            jax.ShapeDtypeStruct((_G, _AD), _F32),
        ],
    )(ag2, p0, p1, af, W_n, U_n, b_n)


def _state_body(asum_ref, esum_ref, st_ref, Ws_ref, Us_ref, bs_ref, out_ref):
    st = st_ref[...]
    W = Ws_ref[...]
    mx = (jnp.dot(asum_ref[...], W[0:16], preferred_element_type=_F32)
          + jnp.dot(esum_ref[...], W[16:32], preferred_element_type=_F32)
          + jnp.dot(st, W[32:40], preferred_element_type=_F32)
          + bs_ref[0:1, :])
    mh = jnp.dot(st, Us_ref[...], preferred_element_type=_F32) + bs_ref[1:2, :]
    out_ref[...] = _gru_math(mx, mh, st)


def _state_call(asum, esum, st, W_s, U_s, b_s):
    full = lambda r, c: pl.BlockSpec((r, c), lambda: (0, 0))
    return pl.pallas_call(
        _state_body,
        in_specs=[
            full(_G, _AD), full(_G, _ED), full(_G, _SD),
            full(_AD + _ED + _SD, 3 * _SD), full(_SD, 3 * _SD),
            full(2, 3 * _SD),
        ],
        out_specs=full(_G, _SD),
        out_shape=jax.ShapeDtypeStruct((_G, _SD), _F32),
    )(asum, esum, st, W_s, U_s, b_s)


def kernel(atom_features, edges_features, state_attrs, pair_indices,
           atom_graph_indices, bond_graph_indices, W_e, U_e, b_e,
           W_n, U_n, b_n, W_s, U_s, b_s, nodes_kernel, nodes_bias):
    src = jnp.asarray(pair_indices[:, 0])
    dst = jnp.asarray(pair_indices[:, 1])
    bg2 = bond_graph_indices.reshape(_E, 1)
    ag_pad = jnp.pad(atom_graph_indices, (0, _NPAD - _N))
    ag2 = ag_pad.reshape(_NPAD, 1)

    nb1 = nodes_bias.reshape(1, _D * _D)
    tile = jnp.tile(jnp.eye(_D, dtype=_F32), (1, _D))
    sel = jnp.kron(jnp.eye(_D, dtype=_F32), jnp.ones((_D, 1), _F32))

    zrows = jnp.zeros((_NSH, _D), _F32)

    af = jnp.pad(atom_features, ((0, _NPAD - _N), (0, 0)))
    ef = edges_features
    st = state_attrs

    for _ in range(_STEPS):
        amsf = _prep_call(ag2, af, st)
        a0, nm = _sc_gather(src, dst, af, amsf)
        ef, trans, e_sum = _edge_call(bg2, a0, nm, ef, st, W_e, U_e, b_e,
                                      nodes_kernel, nb1, tile, sel)
        partials = _sc_scatter(src, trans, zrows)
        af, a_sum = _node_call(ag2, partials[0], partials[1], af, W_n, U_n, b_n)
        st = _state_call(a_sum, e_sum, st, W_s, U_s, b_s)

    return (af[:_N], ef, st)


# speedup vs baseline: 2.2331x; 1.3308x over previous
import functools

import jax
import jax.numpy as jnp
from jax import lax
from jax.experimental import pallas as pl
from jax.experimental.pallas import tpu as pltpu
from jax.experimental.pallas import tpu_sc as plsc

_N = 10000
_E = 160000
_G = 128
_AD = 16
_ED = 16
_SD = 8
_D = _AD + _SD
_STEPS = 2

_NPAD = 10240
_BN = 1024
_BE = 2000

_NW = 32
_EPW = _E // _NW
_KCH = 1000
_NSH = _NPAD // 16

_F32 = jnp.float32


def _gru_math(mx, mh, h):
    u = mx.shape[-1] // 3
    z = jax.nn.sigmoid(mx[:, :u] + mh[:, :u])
    r = jax.nn.sigmoid(mx[:, u:2 * u] + mh[:, u:2 * u])
    hh = jnp.tanh(mx[:, 2 * u:] + r * mh[:, 2 * u:])
    return z * h + (1.0 - z) * hh


def _sc_gather(src, dst, af, amsf):
    mesh = plsc.VectorSubcoreMesh(core_axis_name="c", subcore_axis_name="s")

    @functools.partial(
        pl.kernel,
        out_type=[
            jax.ShapeDtypeStruct((_E, _AD), _F32),
            jax.ShapeDtypeStruct((_E, _D), _F32),
        ],
        mesh=mesh,
        scratch_types=[
            pltpu.VMEM((_KCH,), jnp.int32),
            pltpu.VMEM((_KCH,), jnp.int32),
            pltpu.VMEM((_KCH, _AD), _F32),
            pltpu.VMEM((_KCH, _D), _F32),
            pltpu.SemaphoreType.DMA,
            pltpu.SemaphoreType.DMA,
        ],
        compiler_params=pltpu.CompilerParams(use_tc_tiling_on_sc=False),
    )
    def run(src_h, dst_h, af_h, amsf_h, a0_h, nm_h,
            isrc, idst, abuf, nbuf, sem_a, sem_n):
        wid = lax.axis_index("s") * 2 + lax.axis_index("c")
        base0 = wid * _EPW
        for t in range(_EPW // _KCH):
            base = base0 + t * _KCH
            pltpu.sync_copy(src_h.at[pl.ds(base, _KCH)], isrc)
            pltpu.sync_copy(dst_h.at[pl.ds(base, _KCH)], idst)
            cp_a = pltpu.async_copy(af_h.at[isrc], abuf, sem_a)
            cp_n = pltpu.async_copy(amsf_h.at[idst], nbuf, sem_n)
            cp_a.wait()
            cp_n.wait()
            pltpu.sync_copy(abuf, a0_h.at[pl.ds(base, _KCH)])
            pltpu.sync_copy(nbuf, nm_h.at[pl.ds(base, _KCH)])

    return run(src, dst, af, amsf)


def _sc_scatter(src, trans, zrows):
    mesh = plsc.VectorSubcoreMesh(core_axis_name="c", subcore_axis_name="s")

    @functools.partial(
        pl.kernel,
        out_type=jax.ShapeDtypeStruct((2, _NPAD, _D), _F32),
        mesh=mesh,
        scratch_types=[
            pltpu.VMEM((_KCH,), jnp.int32),
            pltpu.VMEM((_KCH, _D), _F32),
            pltpu.VMEM_SHARED((_NPAD, _D), _F32),
        ],
        compiler_params=pltpu.CompilerParams(use_tc_tiling_on_sc=False),
    )
    def run(src_h, tr_h, z_h, out_h, ibuf, tbuf, acc):
        c = lax.axis_index("c")
        s = lax.axis_index("s")
        pltpu.sync_copy(z_h, acc.at[pl.ds(s * _NSH, _NSH)])
        plsc.subcore_barrier()
        base0 = (c * 16 + s) * _EPW
        for t in range(_EPW // _KCH):
            base = base0 + t * _KCH
            pltpu.sync_copy(src_h.at[pl.ds(base, _KCH)], ibuf)
            pltpu.sync_copy(tr_h.at[pl.ds(base, _KCH)], tbuf)
            pltpu.sync_copy(tbuf, acc.at[ibuf], add=True)
        plsc.subcore_barrier()
        pltpu.sync_copy(acc.at[pl.ds(s * _NSH, _NSH)],
                        out_h.at[c, pl.ds(s * _NSH, _NSH)])

    return run(src, trans, zrows)


def _prep_body(ag_ref, af_ref, st_ref, out_ref):
    oh = (ag_ref[...] == lax.broadcasted_iota(jnp.int32, (1, _G), 1))
    strep = jnp.dot(oh.astype(_F32), st_ref[...], preferred_element_type=_F32, precision=lax.Precision.HIGHEST)
    out_ref[:, :_AD] = af_ref[...]
    out_ref[:, _AD:] = strep


def _prep_call(ag2, af, st):
    return pl.pallas_call(
        _prep_body,
        grid=(_NPAD // _BN,),
        in_specs=[
            pl.BlockSpec((_BN, 1), lambda i: (i, 0)),
            pl.BlockSpec((_BN, _AD), lambda i: (i, 0)),
            pl.BlockSpec((_G, _SD), lambda i: (0, 0)),
        ],
        out_specs=pl.BlockSpec((_BN, _D), lambda i: (i, 0)),
        out_shape=jax.ShapeDtypeStruct((_NPAD, _D), _F32),
    )(ag2, af, st)


def _edge_body(bg_ref, a0_ref, nm_ref, ef_ref, st_ref, We_ref, Ue_ref,
               be_ref, nk_ref, nb_ref, tile_ref, sel_ref,
               efo_ref, tro_ref, esum_ref):
    oh = (bg_ref[...] == lax.broadcasted_iota(jnp.int32, (1, _G), 1))
    oh = oh.astype(_F32)
    st_e = jnp.dot(oh, st_ref[...], preferred_element_type=_F32, precision=lax.Precision.HIGHEST)
    a0 = a0_ref[...]
    nm = nm_ref[...]
    ef = ef_ref[...]
    a1 = nm[:, :_AD]
    W = We_ref[...]
    mx = (jnp.dot(a0, W[0:16], preferred_element_type=_F32)
          + jnp.dot(a1, W[16:32], preferred_element_type=_F32)
          + jnp.dot(st_e, W[32:40], preferred_element_type=_F32)
          + jnp.dot(ef, W[40:56], preferred_element_type=_F32)
          + be_ref[0:1, :])
    mh = jnp.dot(ef, Ue_ref[...], preferred_element_type=_F32) + be_ref[1:2, :]
    ef_new = _gru_math(mx, mh, ef)
    efo_ref[...] = ef_new
    ew = jnp.dot(ef_new, nk_ref[...], preferred_element_type=_F32) + nb_ref[...]
    nt = jnp.dot(nm, tile_ref[...], preferred_element_type=_F32,
                 precision=lax.Precision.HIGHEST)
    tr = jnp.dot(ew * nt, sel_ref[...], preferred_element_type=_F32,
                 precision=lax.Precision.HIGHEST)
    tro_ref[...] = tr
    part = lax.dot_general(oh, ef_new, (((0,), (0,)), ((), ())),
                           preferred_element_type=_F32, precision=lax.Precision.HIGHEST)

    @pl.when(pl.program_id(0) == 0)
    def _():
        esum_ref[...] = jnp.zeros_like(esum_ref)

    esum_ref[...] += part


def _edge_call(bg2, a0, nm, ef, st, W_e, U_e, b_e, nk, nb1, tile, sel):
    full = lambda r, c: pl.BlockSpec((r, c), lambda i: (0, 0))
    return pl.pallas_call(
        _edge_body,
        grid=(_E // _BE,),
        in_specs=[
            pl.BlockSpec((_BE, 1), lambda i: (i, 0)),
            pl.BlockSpec((_BE, _AD), lambda i: (i, 0)),
            pl.BlockSpec((_BE, _D), lambda i: (i, 0)),
            pl.BlockSpec((_BE, _ED), lambda i: (i, 0)),
            full(_G, _SD),
            full(2 * _AD + _SD + _ED, 3 * _ED),
            full(_ED, 3 * _ED),
            full(2, 3 * _ED),
            full(_ED, _D * _D),
            full(1, _D * _D),
            full(_D, _D * _D),
            full(_D * _D, _D),
        ],
        out_specs=[
            pl.BlockSpec((_BE, _ED), lambda i: (i, 0)),
            pl.BlockSpec((_BE, _D), lambda i: (i, 0)),
            pl.BlockSpec((_G, _ED), lambda i: (0, 0)),
        ],
        out_shape=[
            jax.ShapeDtypeStruct((_E, _ED), _F32),
            jax.ShapeDtypeStruct((_E, _D), _F32),
            jax.ShapeDtypeStruct((_G, _ED), _F32),
        ],
    )(bg2, a0, nm, ef, st, W_e, U_e, b_e, nk, nb1, tile, sel)


def _node_body(ag_ref, p0_ref, p1_ref, af_ref, Wn_ref, Un_ref, bn_ref,
               afo_ref, asum_ref):
    agg = p0_ref[...] + p1_ref[...]
    af = af_ref[...]
    mx = jnp.dot(agg, Wn_ref[...], preferred_element_type=_F32) + bn_ref[0:1, :]
    mh = jnp.dot(af, Un_ref[...], preferred_element_type=_F32) + bn_ref[1:2, :]
    af_new = _gru_math(mx, mh, af)
    afo_ref[...] = af_new
    oh = (ag_ref[...] == lax.broadcasted_iota(jnp.int32, (1, _G), 1))
    part = lax.dot_general(oh.astype(_F32), af_new, (((0,), (0,)), ((), ())),
                           preferred_element_type=_F32, precision=lax.Precision.HIGHEST)

    @pl.when(pl.program_id(0) == 0)
    def _():
        asum_ref[...] = jnp.zeros_like(asum_ref)

    asum_ref[...] += part


def _node_call(ag2, p0, p1, af, W_n, U_n, b_n):
    full = lambda r, c: pl.BlockSpec((r, c), lambda i: (0, 0))
    return pl.pallas_call(
        _node_body,
        grid=(_NPAD // _BN,),
        in_specs=[
            pl.BlockSpec((_BN, 1), lambda i: (i, 0)),
            pl.BlockSpec((_BN, _D), lambda i: (i, 0)),
            pl.BlockSpec((_BN, _D), lambda i: (i, 0)),
            pl.BlockSpec((_BN, _AD), lambda i: (i, 0)),
            full(_D, 3 * _AD),
            full(_AD, 3 * _AD),
            full(2, 3 * _AD),
        ],
        out_specs=[
            pl.BlockSpec((_BN, _AD), lambda i: (i, 0)),
            pl.BlockSpec((_G, _AD), lambda i: (0, 0)),
        ],
        out_shape=[
            jax.ShapeDtypeStruct((_NPAD, _AD), _F32),
            jax.ShapeDtypeStruct((_G, _AD), _F32),
        ],
    )(ag2, p0, p1, af, W_n, U_n, b_n)


def _state_body(asum_ref, esum_ref, st_ref, Ws_ref, Us_ref, bs_ref, out_ref):
    st = st_ref[...]
    W = Ws_ref[...]
    mx = (jnp.dot(asum_ref[...], W[0:16], preferred_element_type=_F32)
          + jnp.dot(esum_ref[...], W[16:32], preferred_element_type=_F32)
          + jnp.dot(st, W[32:40], preferred_element_type=_F32)
          + bs_ref[0:1, :])
    mh = jnp.dot(st, Us_ref[...], preferred_element_type=_F32) + bs_ref[1:2, :]
    out_ref[...] = _gru_math(mx, mh, st)


def _state_call(asum, esum, st, W_s, U_s, b_s):
    full = lambda r, c: pl.BlockSpec((r, c), lambda: (0, 0))
    return pl.pallas_call(
        _state_body,
        in_specs=[
            full(_G, _AD), full(_G, _ED), full(_G, _SD),
            full(_AD + _ED + _SD, 3 * _SD), full(_SD, 3 * _SD),
            full(2, 3 * _SD),
        ],
        out_specs=full(_G, _SD),
        out_shape=jax.ShapeDtypeStruct((_G, _SD), _F32),
    )(asum, esum, st, W_s, U_s, b_s)


def kernel(atom_features, edges_features, state_attrs, pair_indices,
           atom_graph_indices, bond_graph_indices, W_e, U_e, b_e,
           W_n, U_n, b_n, W_s, U_s, b_s, nodes_kernel, nodes_bias):
    src = jnp.asarray(pair_indices[:, 0])
    dst = jnp.asarray(pair_indices[:, 1])
    bg2 = bond_graph_indices.reshape(_E, 1)
    ag_pad = jnp.pad(atom_graph_indices, (0, _NPAD - _N))
    ag2 = ag_pad.reshape(_NPAD, 1)

    nb1 = nodes_bias.reshape(1, _D * _D)
    tile = jnp.tile(jnp.eye(_D, dtype=_F32), (1, _D))
    sel = jnp.kron(jnp.eye(_D, dtype=_F32), jnp.ones((_D, 1), _F32))

    zrows = jnp.zeros((_NSH, _D), _F32)

    af = jnp.pad(atom_features, ((0, _NPAD - _N), (0, 0)))
    ef = edges_features
    st = state_attrs

    for _ in range(_STEPS):
        amsf = _prep_call(ag2, af, st)
        a0, nm = _sc_gather(src, dst, af, amsf)
        ef, trans, e_sum = _edge_call(bg2, a0, nm, ef, st, W_e, U_e, b_e,
                                      nodes_kernel, nb1, tile, sel)
        partials = _sc_scatter(src, trans, zrows)
        af, a_sum = _node_call(ag2, partials[0], partials[1], af, W_n, U_n, b_n)
        st = _state_call(a_sum, e_sum, st, W_s, U_s, b_s)

    return (af[:_N], ef, st)


# speedup vs baseline: 3.0009x; 1.3438x over previous
import functools

import jax
import jax.numpy as jnp
from jax import lax
from jax.experimental import pallas as pl
from jax.experimental.pallas import tpu as pltpu
from jax.experimental.pallas import tpu_sc as plsc

_N = 10000
_E = 160000
_G = 128
_AD = 16
_ED = 16
_SD = 8
_D = _AD + _SD
_STEPS = 2

_NPAD = 10240
_BN = 1024
_BE = 3200

_NW = 32
_EPW = _E // _NW
_KCH = 1000
_NSH = _NPAD // 16

_F32 = jnp.float32


def _gru_math(mx, mh, h):
    u = mx.shape[-1] // 3
    z = jax.nn.sigmoid(mx[:, :u] + mh[:, :u])
    r = jax.nn.sigmoid(mx[:, u:2 * u] + mh[:, u:2 * u])
    hh = jnp.tanh(mx[:, 2 * u:] + r * mh[:, 2 * u:])
    return z * h + (1.0 - z) * hh


def _sc_gather(src, dst, af, amsf):
    mesh = plsc.VectorSubcoreMesh(core_axis_name="c", subcore_axis_name="s")

    @functools.partial(
        pl.kernel,
        out_type=[
            jax.ShapeDtypeStruct((_E, _AD), _F32),
            jax.ShapeDtypeStruct((_E, _D), _F32),
        ],
        mesh=mesh,
        scratch_types=[
            pltpu.VMEM((_KCH,), jnp.int32),
            pltpu.VMEM((_KCH,), jnp.int32),
            pltpu.VMEM((_KCH, _AD), _F32),
            pltpu.VMEM((_KCH, _D), _F32),
            pltpu.SemaphoreType.DMA,
            pltpu.SemaphoreType.DMA,
        ],
        compiler_params=pltpu.CompilerParams(use_tc_tiling_on_sc=False),
    )
    def run(src_h, dst_h, af_h, amsf_h, a0_h, nm_h,
            isrc, idst, abuf, nbuf, sem_a, sem_n):
        wid = lax.axis_index("s") * 2 + lax.axis_index("c")
        base0 = wid * _EPW
        for t in range(_EPW // _KCH):
            base = base0 + t * _KCH
            pltpu.sync_copy(src_h.at[pl.ds(base, _KCH)], isrc)
            pltpu.sync_copy(dst_h.at[pl.ds(base, _KCH)], idst)
            cp_a = pltpu.async_copy(af_h.at[isrc], abuf, sem_a)
            cp_n = pltpu.async_copy(amsf_h.at[idst], nbuf, sem_n)
            cp_a.wait()
            cp_n.wait()
            pltpu.sync_copy(abuf, a0_h.at[pl.ds(base, _KCH)])
            pltpu.sync_copy(nbuf, nm_h.at[pl.ds(base, _KCH)])

    return run(src, dst, af, amsf)


def _sc_scatter(src, trans, zrows):
    mesh = plsc.VectorSubcoreMesh(core_axis_name="c", subcore_axis_name="s")

    @functools.partial(
        pl.kernel,
        out_type=jax.ShapeDtypeStruct((2, _NPAD, _D), _F32),
        mesh=mesh,
        scratch_types=[
            pltpu.VMEM((_KCH,), jnp.int32),
            pltpu.VMEM((_KCH, _D), _F32),
            pltpu.VMEM_SHARED((_NPAD, _D), _F32),
        ],
        compiler_params=pltpu.CompilerParams(use_tc_tiling_on_sc=False),
    )
    def run(src_h, tr_h, z_h, out_h, ibuf, tbuf, acc):
        c = lax.axis_index("c")
        s = lax.axis_index("s")
        pltpu.sync_copy(z_h, acc.at[pl.ds(s * _NSH, _NSH)])
        plsc.subcore_barrier()
        base0 = (c * 16 + s) * _EPW
        for t in range(_EPW // _KCH):
            base = base0 + t * _KCH
            pltpu.sync_copy(src_h.at[pl.ds(base, _KCH)], ibuf)
            pltpu.sync_copy(tr_h.at[pl.ds(base, _KCH)], tbuf)
            pltpu.sync_copy(tbuf, acc.at[ibuf], add=True)
        plsc.subcore_barrier()
        pltpu.sync_copy(acc.at[pl.ds(s * _NSH, _NSH)],
                        out_h.at[c, pl.ds(s * _NSH, _NSH)])

    return run(src, trans, zrows)


def _prep_body(ag_ref, af_ref, st_ref, out_ref):
    oh = (ag_ref[...] == lax.broadcasted_iota(jnp.int32, (1, _G), 1))
    strep = jnp.dot(oh.astype(_F32), st_ref[...], preferred_element_type=_F32, precision=lax.Precision.HIGHEST)
    out_ref[:, :_AD] = af_ref[...]
    out_ref[:, _AD:] = strep


def _prep_call(ag2, af, st):
    return pl.pallas_call(
        _prep_body,
        grid=(_NPAD // _BN,),
        in_specs=[
            pl.BlockSpec((_BN, 1), lambda i: (i, 0)),
            pl.BlockSpec((_BN, _AD), lambda i: (i, 0)),
            pl.BlockSpec((_G, _SD), lambda i: (0, 0)),
        ],
        out_specs=pl.BlockSpec((_BN, _D), lambda i: (i, 0)),
        out_shape=jax.ShapeDtypeStruct((_NPAD, _D), _F32),
    )(ag2, af, st)


def _edge_body(bg_ref, a0_ref, nm_ref, ef_ref, st_ref,
               bda0_ref, bda1_ref, bdst_ref, bdef_ref, bdu_ref,
               bx_ref, bh_ref, bdnk_ref, nbw_ref, exj_ref, selw_ref,
               efo_ref, tro_ref, esum_ref):
    a0w = a0_ref[...]
    nmw = nm_ref[...]
    efw = ef_ref[...]
    bg8 = bg_ref[...]
    st = st_ref[...]
    ohs = [(bg8[:, p:p + 1] == lax.broadcasted_iota(jnp.int32, (1, _G), 1))
           .astype(_F32) for p in range(8)]
    stew = jnp.concatenate(
        [jnp.dot(oh, st, preferred_element_type=_F32,
                 precision=lax.Precision.HIGHEST) for oh in ohs], axis=1)
    mx = (jnp.dot(a0w, bda0_ref[...], preferred_element_type=_F32)
          + jnp.dot(nmw, bda1_ref[...], preferred_element_type=_F32)
          + jnp.dot(stew, bdst_ref[...], preferred_element_type=_F32)
          + jnp.dot(efw, bdef_ref[...], preferred_element_type=_F32)
          + bx_ref[...])
    mh = jnp.dot(efw, bdu_ref[...], preferred_element_type=_F32) + bh_ref[...]
    ef_new = _gru_math(mx, mh, efw)
    efo_ref[...] = ef_new
    ew = jnp.dot(ef_new, bdnk_ref[...], preferred_element_type=_F32) + nbw_ref[...]
    nmbig = jnp.dot(nmw, exj_ref[...], preferred_element_type=_F32,
                    precision=lax.Precision.HIGHEST)
    tro_ref[...] = jnp.dot(ew * nmbig, selw_ref[...],
                           preferred_element_type=_F32,
                           precision=lax.Precision.HIGHEST)
    parts = []
    for p in range(8):
        parts.append(lax.dot_general(
            ohs[p], ef_new[:, p * 16:(p + 1) * 16], (((0,), (0,)), ((), ())),
            preferred_element_type=_F32, precision=lax.Precision.HIGHEST))
    part = parts[0]
    for q in parts[1:]:
        part = part + q

    @pl.when(pl.program_id(0) == 0)
    def _():
        esum_ref[...] = jnp.zeros_like(esum_ref)

    esum_ref[...] += part


def _edge_call(bg8, a0w, nmw, efw, st, consts):
    bda0, bda1, bdst, bdef, bdu, bx, bh, bdnk, nbw, exj, selw = consts
    BB = _BE // 8
    EE = _E // 8
    full = lambda r, c: pl.BlockSpec((r, c), lambda i: (0, 0))
    return pl.pallas_call(
        _edge_body,
        grid=(_E // _BE,),
        in_specs=[
            pl.BlockSpec((BB, 8), lambda i: (i, 0)),
            pl.BlockSpec((BB, 128), lambda i: (i, 0)),
            pl.BlockSpec((BB, 192), lambda i: (i, 0)),
            pl.BlockSpec((BB, 128), lambda i: (i, 0)),
            full(_G, _SD),
            full(128, 384), full(192, 384), full(64, 384), full(128, 384),
            full(128, 384), full(1, 384), full(1, 384),
            full(128, 4608), full(1, 4608), full(192, 4608),
            full(4608, 192),
        ],
        out_specs=[
            pl.BlockSpec((BB, 128), lambda i: (i, 0)),
            pl.BlockSpec((BB, 192), lambda i: (i, 0)),
            pl.BlockSpec((_G, _ED), lambda i: (0, 0)),
        ],
        out_shape=[
            jax.ShapeDtypeStruct((EE, 128), _F32),
            jax.ShapeDtypeStruct((EE, 192), _F32),
            jax.ShapeDtypeStruct((_G, _ED), _F32),
        ],
    )(bg8, a0w, nmw, efw, st, bda0, bda1, bdst, bdef, bdu, bx, bh,
      bdnk, nbw, exj, selw)


def _node_body(ag_ref, p0_ref, p1_ref, af_ref, Wn_ref, Un_ref, bn_ref,
               afo_ref, asum_ref):
    agg = p0_ref[...] + p1_ref[...]
    af = af_ref[...]
    mx = jnp.dot(agg, Wn_ref[...], preferred_element_type=_F32) + bn_ref[0:1, :]
    mh = jnp.dot(af, Un_ref[...], preferred_element_type=_F32) + bn_ref[1:2, :]
    af_new = _gru_math(mx, mh, af)
    afo_ref[...] = af_new
    oh = (ag_ref[...] == lax.broadcasted_iota(jnp.int32, (1, _G), 1))
    part = lax.dot_general(oh.astype(_F32), af_new, (((0,), (0,)), ((), ())),
                           preferred_element_type=_F32, precision=lax.Precision.HIGHEST)

    @pl.when(pl.program_id(0) == 0)
    def _():
        asum_ref[...] = jnp.zeros_like(asum_ref)

    asum_ref[...] += part


def _node_call(ag2, p0, p1, af, W_n, U_n, b_n):
    full = lambda r, c: pl.BlockSpec((r, c), lambda i: (0, 0))
    return pl.pallas_call(
        _node_body,
        grid=(_NPAD // _BN,),
        in_specs=[
            pl.BlockSpec((_BN, 1), lambda i: (i, 0)),
            pl.BlockSpec((_BN, _D), lambda i: (i, 0)),
            pl.BlockSpec((_BN, _D), lambda i: (i, 0)),
            pl.BlockSpec((_BN, _AD), lambda i: (i, 0)),
            full(_D, 3 * _AD),
            full(_AD, 3 * _AD),
            full(2, 3 * _AD),
        ],
        out_specs=[
            pl.BlockSpec((_BN, _AD), lambda i: (i, 0)),
            pl.BlockSpec((_G, _AD), lambda i: (0, 0)),
        ],
        out_shape=[
            jax.ShapeDtypeStruct((_NPAD, _AD), _F32),
            jax.ShapeDtypeStruct((_G, _AD), _F32),
        ],
    )(ag2, p0, p1, af, W_n, U_n, b_n)


def _state_body(asum_ref, esum_ref, st_ref, Ws_ref, Us_ref, bs_ref, out_ref):
    st = st_ref[...]
    W = Ws_ref[...]
    mx = (jnp.dot(asum_ref[...], W[0:16], preferred_element_type=_F32)
          + jnp.dot(esum_ref[...], W[16:32], preferred_element_type=_F32)
          + jnp.dot(st, W[32:40], preferred_element_type=_F32)
          + bs_ref[0:1, :])
    mh = jnp.dot(st, Us_ref[...], preferred_element_type=_F32) + bs_ref[1:2, :]
    out_ref[...] = _gru_math(mx, mh, st)


def _state_call(asum, esum, st, W_s, U_s, b_s):
    full = lambda r, c: pl.BlockSpec((r, c), lambda: (0, 0))
    return pl.pallas_call(
        _state_body,
        in_specs=[
            full(_G, _AD), full(_G, _ED), full(_G, _SD),
            full(_AD + _ED + _SD, 3 * _SD), full(_SD, 3 * _SD),
            full(2, 3 * _SD),
        ],
        out_specs=full(_G, _SD),
        out_shape=jax.ShapeDtypeStruct((_G, _SD), _F32),
    )(asum, esum, st, W_s, U_s, b_s)


def kernel(atom_features, edges_features, state_attrs, pair_indices,
           atom_graph_indices, bond_graph_indices, W_e, U_e, b_e,
           W_n, U_n, b_n, W_s, U_s, b_s, nodes_kernel, nodes_bias):
    src = jnp.asarray(pair_indices[:, 0])
    dst = jnp.asarray(pair_indices[:, 1])
    bg8 = bond_graph_indices.reshape(_E // 8, 8)
    ag_pad = jnp.pad(atom_graph_indices, (0, _NPAD - _N))
    ag2 = ag_pad.reshape(_NPAD, 1)

    I8 = jnp.eye(8, dtype=_F32)

    def kron8(m):
        return jnp.kron(I8, m)

    gates = [slice(16 * g, 16 * (g + 1)) for g in range(3)]
    bda0 = jnp.concatenate([kron8(W_e[0:16, cg]) for cg in gates], axis=1)
    a1m = [jnp.concatenate([W_e[16:32, cg], jnp.zeros((8, 16), _F32)], axis=0)
           for cg in gates]
    bda1 = jnp.concatenate([kron8(m) for m in a1m], axis=1)
    bdst = jnp.concatenate([kron8(W_e[32:40, cg]) for cg in gates], axis=1)
    bdef = jnp.concatenate([kron8(W_e[40:56, cg]) for cg in gates], axis=1)
    bdu = jnp.concatenate([kron8(U_e[:, cg]) for cg in gates], axis=1)
    bx = jnp.concatenate([jnp.tile(b_e[0:1, cg], (1, 8)) for cg in gates], axis=1)
    bh = jnp.concatenate([jnp.tile(b_e[1:2, cg], (1, 8)) for cg in gates], axis=1)

    bdnk = kron8(nodes_kernel)
    nbw = jnp.tile(nodes_bias.reshape(1, _D * _D), (1, 8))
    exj = kron8(jnp.tile(jnp.eye(_D, dtype=_F32), (1, _D)))
    selw = kron8(jnp.kron(jnp.eye(_D, dtype=_F32),
                          jnp.ones((_D, 1), _F32)))
    consts = (bda0, bda1, bdst, bdef, bdu, bx, bh, bdnk, nbw, exj, selw)

    zrows = jnp.zeros((_NSH, _D), _F32)

    af = jnp.pad(atom_features, ((0, _NPAD - _N), (0, 0)))
    efw = edges_features.reshape(_E // 8, 128)
    st = state_attrs

    for _ in range(_STEPS):
        amsf = _prep_call(ag2, af, st)
        a0, nm = _sc_gather(src, dst, af, amsf)
        a0w = a0.reshape(_E // 8, 128)
        nmw = nm.reshape(_E // 8, 192)
        efw, trow, e_sum = _edge_call(bg8, a0w, nmw, efw, st, consts)
        trans = trow.reshape(_E, _D)
        partials = _sc_scatter(src, trans, zrows)
        af, a_sum = _node_call(ag2, partials[0], partials[1], af, W_n, U_n, b_n)
        st = _state_call(a_sum, e_sum, st, W_s, U_s, b_s)

    return (af[:_N], efw.reshape(_E, _ED), st)
